# baseline (speedup 1.0000x reference)
import functools

import jax
import jax.numpy as jnp
from jax import lax
from jax.experimental import pallas as pl
from jax.experimental.pallas import tpu as pltpu
from jax.experimental.pallas import tpu_sc as plsc

N = 10000
E = 160000
D = 256
DH = 128
NC = 2
NS = 16
CH = 100

NPT = N // NS
NROW = 125
NCH2 = E // NS // CH
RB2 = 10
GD2 = 7
assert (NCH2 - RB2) % RB2 == 0 and NCH2 % RB2 == 0
NCH0 = E // (NC * NS) // CH


def _deg_body(dst_hbm, zeros_hbm, ones_hbm, out_hbm, acc_sh, didx_v, ones_v,
              zb_v, sem):
  c = lax.axis_index("c")
  s = lax.axis_index("s")

  pltpu.sync_copy(zeros_hbm, zb_v)
  for j in range(NPT // NROW):
    pltpu.sync_copy(zb_v, acc_sh.at[pl.ds(s * NPT + j * NROW, NROW)])
  pltpu.sync_copy(ones_hbm, ones_v)
  tile = c * NS + s
  pltpu.sync_copy(dst_hbm.at[pl.ds(tile * NCH0, NCH0)], didx_v)
  plsc.subcore_barrier()

  def body(k, carry):
    for b in range(2):
      pltpu.async_copy(ones_v, acc_sh.at[didx_v.at[2 * k + b]], sem,
                       add=True)
    for b in range(2):
      pltpu.make_async_copy(ones_v, acc_sh.at[didx_v.at[2 * k + b]],
                            sem).wait()
    return carry

  lax.fori_loop(0, NCH0 // 2, body, 0)
  plsc.subcore_barrier()

  for j in range(NPT // NROW):
    pltpu.sync_copy(acc_sh.at[pl.ds(s * NPT + j * NROW, NROW)], zb_v)
    pltpu.sync_copy(zb_v, out_hbm.at[pl.ds(c * N + s * NPT + j * NROW, NROW)])


@functools.cache
def _deg_call():
  mesh = plsc.VectorSubcoreMesh(
      core_axis_name="c", subcore_axis_name="s",
      num_cores=NC, num_subcores=NS)
  return pl.kernel(
      _deg_body,
      out_type=jax.ShapeDtypeStruct((NC * N, 16), jnp.float32),
      mesh=mesh,
      compiler_params=pltpu.CompilerParams(use_tc_tiling_on_sc=False),
      scratch_types=[
          pltpu.VMEM_SHARED((N, 16), jnp.float32),
          pltpu.VMEM((NCH0, CH), jnp.int32),
          pltpu.VMEM((CH, 16), jnp.float32),
          pltpu.VMEM((NROW, 16), jnp.float32),
          pltpu.SemaphoreType.DMA,
      ],
  )


def _agg_body(src_hbm, dst_hbm, g_hbm, out_hbm, acc_sh, sidx_v, didx_v,
              *rest):
  c = lax.axis_index("c")
  s = lax.axis_index("s")
  bufs = rest[:RB2]
  sg = rest[RB2:2 * RB2]
  ss = rest[2 * RB2:3 * RB2]

  def gather(i, b):
    pltpu.async_copy(g_hbm.at[sidx_v.at[i]], bufs[b], sg[b])

  def gather_wait(i, b):
    pltpu.make_async_copy(g_hbm.at[sidx_v.at[i]], bufs[b], sg[b]).wait()

  def scatter(i, b):
    pltpu.async_copy(bufs[b], acc_sh.at[didx_v.at[i]], ss[b], add=True)

  def scatter_wait(i, b):
    pltpu.make_async_copy(bufs[b], acc_sh.at[didx_v.at[i]], ss[b]).wait()

  for j in range(NPT // CH):
    pltpu.sync_copy(g_hbm.at[pl.ds(c * N + s * NPT + j * CH, CH)], bufs[0])
    pltpu.sync_copy(bufs[0], acc_sh.at[pl.ds(s * NPT + j * CH, CH)])
  tail = NPT - NPT % CH
  pltpu.sync_copy(g_hbm.at[pl.ds(c * N + s * NPT + tail, NPT % CH)],
                  bufs[0].at[pl.ds(0, NPT % CH)])
  pltpu.sync_copy(bufs[0].at[pl.ds(0, NPT % CH)],
                  acc_sh.at[pl.ds(s * NPT + tail, NPT % CH)])

  pltpu.sync_copy(src_hbm.at[pl.ds((c * NS + s) * NCH2, NCH2)], sidx_v)
  pltpu.sync_copy(dst_hbm.at[pl.ds(s * NCH2, NCH2)], didx_v)

  for i in range(GD2):
    gather(i, i)
  plsc.subcore_barrier()

  LAG = RB2 - GD2
  for i in range(LAG):
    gather_wait(i, i)
    scatter(i, i)
    gather(i + GD2, (i + GD2) % RB2)

  def body(k, carry):
    for t in range(RB2):
      i = RB2 * k + t + LAG
      b = (t + LAG) % RB2
      gather_wait(i, b)
      scatter(i, b)
      scatter_wait(i - LAG, t)
      gather(i + GD2, (t + LAG + GD2) % RB2)
    return carry

  lax.fori_loop(0, (NCH2 - RB2) // RB2, body, 0)
  for i in range(NCH2 - GD2, NCH2):
    gather_wait(i, i % RB2)
    scatter(i, i % RB2)
    scatter_wait(i - LAG, (i - LAG) % RB2)
  for i in range(NCH2 - LAG, NCH2):
    scatter_wait(i, i % RB2)
  plsc.subcore_barrier()

  for j in range(NPT // CH):
    pltpu.sync_copy(acc_sh.at[pl.ds(s * NPT + j * CH, CH)], bufs[0])
    pltpu.sync_copy(bufs[0], out_hbm.at[pl.ds(c * N + s * NPT + j * CH, CH)])
  tail = NPT - NPT % CH
  pltpu.sync_copy(acc_sh.at[pl.ds(s * NPT + tail, NPT % CH)],
                  bufs[0].at[pl.ds(0, NPT % CH)])
  pltpu.sync_copy(bufs[0].at[pl.ds(0, NPT % CH)],
                  out_hbm.at[pl.ds(c * N + s * NPT + tail, NPT % CH)])


@functools.cache
def _agg_call():
  mesh = plsc.VectorSubcoreMesh(
      core_axis_name="c", subcore_axis_name="s",
      num_cores=NC, num_subcores=NS)
  return pl.kernel(
      _agg_body,
      out_type=jax.ShapeDtypeStruct((NC * N, DH), jnp.bfloat16),
      mesh=mesh,
      compiler_params=pltpu.CompilerParams(use_tc_tiling_on_sc=False),
      scratch_types=(
          [pltpu.VMEM_SHARED((N, DH), jnp.bfloat16)]
          + [pltpu.VMEM((NCH2, CH), jnp.int32)] * 2
          + [pltpu.VMEM((CH, DH), jnp.bfloat16)] * RB2
          + [pltpu.SemaphoreType.DMA] * (2 * RB2)
      ),
  )


_RB = 1000


def _mm_body(x_ref, wg_ref, wl_ref, b_ref, h_ref, linb_ref):
  xb = x_ref[...]
  h_ref[...] = jnp.dot(xb, wg_ref[...], preferred_element_type=jnp.float32)
  linb_ref[...] = (
      jnp.dot(xb, wl_ref[...], preferred_element_type=jnp.float32)
      + b_ref[...])


def _scale_body(h_ref, degp_ref, g_ref):
  deg = degp_ref[0, :, 0:1] + degp_ref[1, :, 0:1] + 1.0
  dinv = lax.rsqrt(deg)
  g = (h_ref[...] * dinv).astype(jnp.bfloat16)
  g_ref[0] = g[:, :DH]
  g_ref[1] = g[:, DH:]


def _epi_body(s_ref, linb_ref, degp_ref, out_ref):
  deg = degp_ref[0, :, 0:1] + degp_ref[1, :, 0:1] + 1.0
  dinv = lax.rsqrt(deg)
  s = jnp.concatenate([s_ref[0], s_ref[1]], axis=1).astype(jnp.float32)
  out_ref[...] = jnp.maximum(s * dinv + linb_ref[...], 0.0)


def kernel(x, edge_index, W_gcn, b_gcn, W_lin, b_lin):
  src = edge_index[0]
  dst = edge_index[1]
  src2 = jnp.concatenate([src, src + N]).reshape(NC * E // CH, CH)
  dst2 = dst.reshape(E // CH, CH)

  degp = _deg_call()(
      dst2,
      jnp.zeros((NROW, 16), jnp.float32),
      jnp.ones((CH, 16), jnp.float32),
  )

  grid = N // _RB
  h, linb = pl.pallas_call(
      _mm_body,
      grid=(grid,),
      in_specs=[
          pl.BlockSpec((_RB, D), lambda i: (i, 0)),
          pl.BlockSpec((D, D), lambda i: (0, 0)),
          pl.BlockSpec((D, D), lambda i: (0, 0)),
          pl.BlockSpec((1, D), lambda i: (0, 0)),
      ],
      out_specs=[
          pl.BlockSpec((_RB, D), lambda i: (i, 0)),
          pl.BlockSpec((_RB, D), lambda i: (i, 0)),
      ],
      out_shape=[
          jax.ShapeDtypeStruct((N, D), jnp.float32),
          jax.ShapeDtypeStruct((N, D), jnp.float32),
      ],
  )(x, W_gcn, W_lin, (b_gcn + b_lin).reshape(1, D))

  g = pl.pallas_call(
      _scale_body,
      grid=(grid,),
      in_specs=[
          pl.BlockSpec((_RB, D), lambda i: (i, 0)),
          pl.BlockSpec((NC, _RB, 16), lambda i: (0, i, 0)),
      ],
      out_specs=pl.BlockSpec((NC, _RB, DH), lambda i: (0, i, 0)),
      out_shape=jax.ShapeDtypeStruct((NC, N, DH), jnp.bfloat16),
  )(h, degp.reshape(NC, N, 16))

  s = _agg_call()(src2, dst2, g.reshape(NC * N, DH))

  out = pl.pallas_call(
      _epi_body,
      grid=(grid,),
      in_specs=[
          pl.BlockSpec((NC, _RB, DH), lambda i: (0, i, 0)),
          pl.BlockSpec((_RB, D), lambda i: (i, 0)),
          pl.BlockSpec((NC, _RB, 16), lambda i: (0, i, 0)),
      ],
      out_specs=pl.BlockSpec((_RB, D), lambda i: (i, 0)),
      out_shape=jax.ShapeDtypeStruct((N, D), jnp.float32),
  )(s.reshape(NC, N, DH), linb, degp.reshape(NC, N, 16))
  return out


# speedup vs baseline: 1.0010x; 1.0010x over previous
import functools

import jax
import jax.numpy as jnp
from jax import lax
from jax.experimental import pallas as pl
from jax.experimental.pallas import tpu as pltpu
from jax.experimental.pallas import tpu_sc as plsc

N = 10000
E = 160000
D = 256
DH = 128
NC = 2
NS = 16
CH = 100

NPT = N // NS
NROW = 125
NCH2 = E // NS // CH
RB2 = 10
GD2 = 5
assert (NCH2 - RB2) % RB2 == 0 and NCH2 % RB2 == 0
NCH0 = E // (NC * NS) // CH


def _deg_body(dst_hbm, zeros_hbm, ones_hbm, out_hbm, acc_sh, didx_v, ones_v,
              zb_v, sem):
  c = lax.axis_index("c")
  s = lax.axis_index("s")

  pltpu.sync_copy(zeros_hbm, zb_v)
  for j in range(NPT // NROW):
    pltpu.sync_copy(zb_v, acc_sh.at[pl.ds(s * NPT + j * NROW, NROW)])
  pltpu.sync_copy(ones_hbm, ones_v)
  tile = c * NS + s
  pltpu.sync_copy(dst_hbm.at[pl.ds(tile * NCH0, NCH0)], didx_v)
  plsc.subcore_barrier()

  def body(k, carry):
    for b in range(2):
      pltpu.async_copy(ones_v, acc_sh.at[didx_v.at[2 * k + b]], sem,
                       add=True)
    for b in range(2):
      pltpu.make_async_copy(ones_v, acc_sh.at[didx_v.at[2 * k + b]],
                            sem).wait()
    return carry

  lax.fori_loop(0, NCH0 // 2, body, 0)
  plsc.subcore_barrier()

  for j in range(NPT // NROW):
    pltpu.sync_copy(acc_sh.at[pl.ds(s * NPT + j * NROW, NROW)], zb_v)
    pltpu.sync_copy(zb_v, out_hbm.at[pl.ds(c * N + s * NPT + j * NROW, NROW)])


@functools.cache
def _deg_call():
  mesh = plsc.VectorSubcoreMesh(
      core_axis_name="c", subcore_axis_name="s",
      num_cores=NC, num_subcores=NS)
  return pl.kernel(
      _deg_body,
      out_type=jax.ShapeDtypeStruct((NC * N, 16), jnp.float32),
      mesh=mesh,
      compiler_params=pltpu.CompilerParams(use_tc_tiling_on_sc=False),
      scratch_types=[
          pltpu.VMEM_SHARED((N, 16), jnp.float32),
          pltpu.VMEM((NCH0, CH), jnp.int32),
          pltpu.VMEM((CH, 16), jnp.float32),
          pltpu.VMEM((NROW, 16), jnp.float32),
          pltpu.SemaphoreType.DMA,
      ],
  )


def _agg_body(src_hbm, dst_hbm, g_hbm, out_hbm, acc_sh, sidx_v, didx_v,
              *rest):
  c = lax.axis_index("c")
  s = lax.axis_index("s")
  bufs = rest[:RB2]
  sg = rest[RB2:2 * RB2]
  ss = rest[2 * RB2:3 * RB2]

  def gather(i, b):
    pltpu.async_copy(g_hbm.at[sidx_v.at[i]], bufs[b], sg[b])

  def gather_wait(i, b):
    pltpu.make_async_copy(g_hbm.at[sidx_v.at[i]], bufs[b], sg[b]).wait()

  def scatter(i, b):
    pltpu.async_copy(bufs[b], acc_sh.at[didx_v.at[i]], ss[b], add=True)

  def scatter_wait(i, b):
    pltpu.make_async_copy(bufs[b], acc_sh.at[didx_v.at[i]], ss[b]).wait()

  for j in range(NPT // CH):
    pltpu.sync_copy(g_hbm.at[pl.ds(c * N + s * NPT + j * CH, CH)], bufs[0])
    pltpu.sync_copy(bufs[0], acc_sh.at[pl.ds(s * NPT + j * CH, CH)])
  tail = NPT - NPT % CH
  pltpu.sync_copy(g_hbm.at[pl.ds(c * N + s * NPT + tail, NPT % CH)],
                  bufs[0].at[pl.ds(0, NPT % CH)])
  pltpu.sync_copy(bufs[0].at[pl.ds(0, NPT % CH)],
                  acc_sh.at[pl.ds(s * NPT + tail, NPT % CH)])

  pltpu.sync_copy(src_hbm.at[pl.ds((c * NS + s) * NCH2, NCH2)], sidx_v)
  pltpu.sync_copy(dst_hbm.at[pl.ds(s * NCH2, NCH2)], didx_v)

  for i in range(GD2):
    gather(i, i)
  plsc.subcore_barrier()

  LAG = RB2 - GD2
  for i in range(LAG):
    gather_wait(i, i)
    scatter(i, i)
    gather(i + GD2, (i + GD2) % RB2)

  def body(k, carry):
    for t in range(RB2):
      i = RB2 * k + t + LAG
      b = (t + LAG) % RB2
      gather_wait(i, b)
      scatter(i, b)
      scatter_wait(i - LAG, t)
      gather(i + GD2, (t + LAG + GD2) % RB2)
    return carry

  lax.fori_loop(0, (NCH2 - RB2) // RB2, body, 0)
  for i in range(NCH2 - GD2, NCH2):
    gather_wait(i, i % RB2)
    scatter(i, i % RB2)
    scatter_wait(i - LAG, (i - LAG) % RB2)
  for i in range(NCH2 - LAG, NCH2):
    scatter_wait(i, i % RB2)
  plsc.subcore_barrier()

  for j in range(NPT // CH):
    pltpu.sync_copy(acc_sh.at[pl.ds(s * NPT + j * CH, CH)], bufs[0])
    pltpu.sync_copy(bufs[0], out_hbm.at[pl.ds(c * N + s * NPT + j * CH, CH)])
  tail = NPT - NPT % CH
  pltpu.sync_copy(acc_sh.at[pl.ds(s * NPT + tail, NPT % CH)],
                  bufs[0].at[pl.ds(0, NPT % CH)])
  pltpu.sync_copy(bufs[0].at[pl.ds(0, NPT % CH)],
                  out_hbm.at[pl.ds(c * N + s * NPT + tail, NPT % CH)])


@functools.cache
def _agg_call():
  mesh = plsc.VectorSubcoreMesh(
      core_axis_name="c", subcore_axis_name="s",
      num_cores=NC, num_subcores=NS)
  return pl.kernel(
      _agg_body,
      out_type=jax.ShapeDtypeStruct((NC * N, DH), jnp.bfloat16),
      mesh=mesh,
      compiler_params=pltpu.CompilerParams(use_tc_tiling_on_sc=False),
      scratch_types=(
          [pltpu.VMEM_SHARED((N, DH), jnp.bfloat16)]
          + [pltpu.VMEM((NCH2, CH), jnp.int32)] * 2
          + [pltpu.VMEM((CH, DH), jnp.bfloat16)] * RB2
          + [pltpu.SemaphoreType.DMA] * (2 * RB2)
      ),
  )


_RB = 1000


def _mm_body(x_ref, wg_ref, wl_ref, b_ref, degp_ref, g_ref, linb_ref):
  xb = x_ref[...]
  deg = degp_ref[0, :, 0:1] + degp_ref[1, :, 0:1] + 1.0
  dinv = lax.rsqrt(deg)
  h = jnp.dot(xb, wg_ref[...], preferred_element_type=jnp.float32)
  g = (h * dinv).astype(jnp.bfloat16)
  g_ref[0] = g[:, :DH]
  g_ref[1] = g[:, DH:]
  linb_ref[...] = (
      jnp.dot(xb, wl_ref[...], preferred_element_type=jnp.float32)
      + b_ref[...])


def _epi_body(s_ref, linb_ref, degp_ref, out_ref):
  deg = degp_ref[0, :, 0:1] + degp_ref[1, :, 0:1] + 1.0
  dinv = lax.rsqrt(deg)
  s = jnp.concatenate([s_ref[0], s_ref[1]], axis=1).astype(jnp.float32)
  out_ref[...] = jnp.maximum(s * dinv + linb_ref[...], 0.0)


def kernel(x, edge_index, W_gcn, b_gcn, W_lin, b_lin):
  src = edge_index[0]
  dst = edge_index[1]
  src2 = jnp.concatenate([src, src + N]).reshape(NC * E // CH, CH)
  dst2 = dst.reshape(E // CH, CH)

  degp = _deg_call()(
      dst2,
      jnp.zeros((NROW, 16), jnp.float32),
      jnp.ones((CH, 16), jnp.float32),
  )

  grid = N // _RB
  g, linb = pl.pallas_call(
      _mm_body,
      grid=(grid,),
      in_specs=[
          pl.BlockSpec((_RB, D), lambda i: (i, 0)),
          pl.BlockSpec((D, D), lambda i: (0, 0)),
          pl.BlockSpec((D, D), lambda i: (0, 0)),
          pl.BlockSpec((1, D), lambda i: (0, 0)),
          pl.BlockSpec((NC, _RB, 16), lambda i: (0, i, 0)),
      ],
      out_specs=[
          pl.BlockSpec((NC, _RB, DH), lambda i: (0, i, 0)),
          pl.BlockSpec((_RB, D), lambda i: (i, 0)),
      ],
      out_shape=[
          jax.ShapeDtypeStruct((NC, N, DH), jnp.bfloat16),
          jax.ShapeDtypeStruct((N, D), jnp.float32),
      ],
  )(x, W_gcn, W_lin, (b_gcn + b_lin).reshape(1, D),
    degp.reshape(NC, N, 16))

  s = _agg_call()(src2, dst2, g.reshape(NC * N, DH))

  out = pl.pallas_call(
      _epi_body,
      grid=(grid,),
      in_specs=[
          pl.BlockSpec((NC, _RB, DH), lambda i: (0, i, 0)),
          pl.BlockSpec((_RB, D), lambda i: (i, 0)),
          pl.BlockSpec((NC, _RB, 16), lambda i: (0, i, 0)),
      ],
      out_specs=pl.BlockSpec((_RB, D), lambda i: (i, 0)),
      out_shape=jax.ShapeDtypeStruct((N, D), jnp.float32),
  )(s.reshape(NC, N, DH), linb, degp.reshape(NC, N, 16))
  return out


# speedup vs baseline: 1.0262x; 1.0251x over previous
import functools

import jax
import jax.numpy as jnp
from jax import lax
from jax.experimental import pallas as pl
from jax.experimental.pallas import tpu as pltpu
from jax.experimental.pallas import tpu_sc as plsc

N = 10000
E = 160000
D = 256
DH = 128
NC = 2
NS = 16
CH = 100

NPT = N // NS
NROW = 125
NCH2 = E // NS // CH
RB2 = 10
GD2 = 8
assert (NCH2 - RB2) % RB2 == 0 and NCH2 % RB2 == 0
NCH0 = E // (NC * NS) // CH


def _deg_body(dst_hbm, zeros_hbm, ones_hbm, out_hbm, acc_sh, didx_v, ones_v,
              zb_v, sem):
  c = lax.axis_index("c")
  s = lax.axis_index("s")

  pltpu.sync_copy(zeros_hbm, zb_v)
  for j in range(NPT // NROW):
    pltpu.sync_copy(zb_v, acc_sh.at[pl.ds(s * NPT + j * NROW, NROW)])
  pltpu.sync_copy(ones_hbm, ones_v)
  tile = c * NS + s
  pltpu.sync_copy(dst_hbm.at[pl.ds(tile * NCH0, NCH0)], didx_v)
  plsc.subcore_barrier()

  def body(k, carry):
    for b in range(2):
      pltpu.async_copy(ones_v, acc_sh.at[didx_v.at[2 * k + b]], sem,
                       add=True)
    for b in range(2):
      pltpu.make_async_copy(ones_v, acc_sh.at[didx_v.at[2 * k + b]],
                            sem).wait()
    return carry

  lax.fori_loop(0, NCH0 // 2, body, 0)
  plsc.subcore_barrier()

  for j in range(NPT // NROW):
    pltpu.sync_copy(acc_sh.at[pl.ds(s * NPT + j * NROW, NROW)], zb_v)
    pltpu.sync_copy(zb_v, out_hbm.at[pl.ds(c * N + s * NPT + j * NROW, NROW)])


@functools.cache
def _deg_call():
  mesh = plsc.VectorSubcoreMesh(
      core_axis_name="c", subcore_axis_name="s",
      num_cores=NC, num_subcores=NS)
  return pl.kernel(
      _deg_body,
      out_type=jax.ShapeDtypeStruct((NC * N, 16), jnp.float32),
      mesh=mesh,
      compiler_params=pltpu.CompilerParams(use_tc_tiling_on_sc=False),
      scratch_types=[
          pltpu.VMEM_SHARED((N, 16), jnp.float32),
          pltpu.VMEM((NCH0, CH), jnp.int32),
          pltpu.VMEM((CH, 16), jnp.float32),
          pltpu.VMEM((NROW, 16), jnp.float32),
          pltpu.SemaphoreType.DMA,
      ],
  )


def _agg_body(src_hbm, dst_hbm, g_hbm, out_hbm, acc_sh, sidx_v, didx_v,
              *rest):
  c = lax.axis_index("c")
  s = lax.axis_index("s")
  bufs = rest[:RB2]
  sg = rest[RB2:2 * RB2]
  ss = rest[2 * RB2:3 * RB2]

  def gather(i, b):
    pltpu.async_copy(g_hbm.at[sidx_v.at[i]], bufs[b], sg[b])

  def gather_wait(i, b):
    pltpu.make_async_copy(g_hbm.at[sidx_v.at[i]], bufs[b], sg[b]).wait()

  def scatter(i, b):
    pltpu.async_copy(bufs[b], acc_sh.at[didx_v.at[i]], ss[b], add=True)

  def scatter_wait(i, b):
    pltpu.make_async_copy(bufs[b], acc_sh.at[didx_v.at[i]], ss[b]).wait()

  for j in range(NPT // CH):
    pltpu.sync_copy(g_hbm.at[pl.ds(c * N + s * NPT + j * CH, CH)], bufs[0])
    pltpu.sync_copy(bufs[0], acc_sh.at[pl.ds(s * NPT + j * CH, CH)])
  tail = NPT - NPT % CH
  pltpu.sync_copy(g_hbm.at[pl.ds(c * N + s * NPT + tail, NPT % CH)],
                  bufs[0].at[pl.ds(0, NPT % CH)])
  pltpu.sync_copy(bufs[0].at[pl.ds(0, NPT % CH)],
                  acc_sh.at[pl.ds(s * NPT + tail, NPT % CH)])

  pltpu.sync_copy(src_hbm.at[pl.ds((c * NS + s) * NCH2, NCH2)], sidx_v)
  pltpu.sync_copy(dst_hbm.at[pl.ds(s * NCH2, NCH2)], didx_v)

  for i in range(GD2):
    gather(i, i)
  plsc.subcore_barrier()

  LAG = RB2 - GD2
  for i in range(LAG):
    gather_wait(i, i)
    scatter(i, i)
    gather(i + GD2, (i + GD2) % RB2)

  def body(k, carry):
    for t in range(RB2):
      i = RB2 * k + t + LAG
      b = (t + LAG) % RB2
      gather_wait(i, b)
      scatter(i, b)
      scatter_wait(i - LAG, t)
      gather(i + GD2, (t + LAG + GD2) % RB2)
    return carry

  lax.fori_loop(0, (NCH2 - RB2) // RB2, body, 0)
  for i in range(NCH2 - GD2, NCH2):
    gather_wait(i, i % RB2)
    scatter(i, i % RB2)
    scatter_wait(i - LAG, (i - LAG) % RB2)
  for i in range(NCH2 - LAG, NCH2):
    scatter_wait(i, i % RB2)
  plsc.subcore_barrier()

  for j in range(NPT // CH):
    pltpu.sync_copy(acc_sh.at[pl.ds(s * NPT + j * CH, CH)], bufs[0])
    pltpu.sync_copy(bufs[0], out_hbm.at[pl.ds(c * N + s * NPT + j * CH, CH)])
  tail = NPT - NPT % CH
  pltpu.sync_copy(acc_sh.at[pl.ds(s * NPT + tail, NPT % CH)],
                  bufs[0].at[pl.ds(0, NPT % CH)])
  pltpu.sync_copy(bufs[0].at[pl.ds(0, NPT % CH)],
                  out_hbm.at[pl.ds(c * N + s * NPT + tail, NPT % CH)])


@functools.cache
def _agg_call():
  mesh = plsc.VectorSubcoreMesh(
      core_axis_name="c", subcore_axis_name="s",
      num_cores=NC, num_subcores=NS)
  return pl.kernel(
      _agg_body,
      out_type=jax.ShapeDtypeStruct((NC * N, DH), jnp.bfloat16),
      mesh=mesh,
      compiler_params=pltpu.CompilerParams(use_tc_tiling_on_sc=False),
      scratch_types=(
          [pltpu.VMEM_SHARED((N, DH), jnp.bfloat16)]
          + [pltpu.VMEM((NCH2, CH), jnp.int32)] * 2
          + [pltpu.VMEM((CH, DH), jnp.bfloat16)] * RB2
          + [pltpu.SemaphoreType.DMA] * (2 * RB2)
      ),
  )


_RB = 1000


def _mm_body(x_ref, wg_ref, wl_ref, b_ref, degp_ref, g_ref, linb_ref):
  xb = x_ref[...]
  deg = degp_ref[0, :, 0:1] + degp_ref[1, :, 0:1] + 1.0
  dinv = lax.rsqrt(deg)
  h = jnp.dot(xb, wg_ref[...], preferred_element_type=jnp.float32)
  g = (h * dinv).astype(jnp.bfloat16)
  g_ref[0] = g[:, :DH]
  g_ref[1] = g[:, DH:]
  linb_ref[...] = (
      jnp.dot(xb, wl_ref[...], preferred_element_type=jnp.float32)
      + b_ref[...])


def _epi_body(s_ref, linb_ref, degp_ref, out_ref):
  deg = degp_ref[0, :, 0:1] + degp_ref[1, :, 0:1] + 1.0
  dinv = lax.rsqrt(deg)
  s = jnp.concatenate([s_ref[0], s_ref[1]], axis=1).astype(jnp.float32)
  out_ref[...] = jnp.maximum(s * dinv + linb_ref[...], 0.0)


def kernel(x, edge_index, W_gcn, b_gcn, W_lin, b_lin):
  src = edge_index[0]
  dst = edge_index[1]
  src2 = jnp.concatenate([src, src + N]).reshape(NC * E // CH, CH)
  dst2 = dst.reshape(E // CH, CH)

  degp = _deg_call()(
      dst2,
      jnp.zeros((NROW, 16), jnp.float32),
      jnp.ones((CH, 16), jnp.float32),
  )

  grid = N // _RB
  g, linb = pl.pallas_call(
      _mm_body,
      grid=(grid,),
      in_specs=[
          pl.BlockSpec((_RB, D), lambda i: (i, 0)),
          pl.BlockSpec((D, D), lambda i: (0, 0)),
          pl.BlockSpec((D, D), lambda i: (0, 0)),
          pl.BlockSpec((1, D), lambda i: (0, 0)),
          pl.BlockSpec((NC, _RB, 16), lambda i: (0, i, 0)),
      ],
      out_specs=[
          pl.BlockSpec((NC, _RB, DH), lambda i: (0, i, 0)),
          pl.BlockSpec((_RB, D), lambda i: (i, 0)),
      ],
      out_shape=[
          jax.ShapeDtypeStruct((NC, N, DH), jnp.bfloat16),
          jax.ShapeDtypeStruct((N, D), jnp.float32),
      ],
  )(x, W_gcn, W_lin, (b_gcn + b_lin).reshape(1, D),
    degp.reshape(NC, N, 16))

  s = _agg_call()(src2, dst2, g.reshape(NC * N, DH))

  out = pl.pallas_call(
      _epi_body,
      grid=(grid,),
      in_specs=[
          pl.BlockSpec((NC, _RB, DH), lambda i: (0, i, 0)),
          pl.BlockSpec((_RB, D), lambda i: (i, 0)),
          pl.BlockSpec((NC, _RB, 16), lambda i: (0, i, 0)),
      ],
      out_specs=pl.BlockSpec((_RB, D), lambda i: (i, 0)),
      out_shape=jax.ShapeDtypeStruct((N, D), jnp.float32),
  )(s.reshape(NC, N, DH), linb, degp.reshape(NC, N, 16))
  return out
